# baseline (speedup 1.0000x reference)
import jax
import jax.numpy as jnp
from jax import lax
from jax.experimental import pallas as pl
from jax.experimental.pallas import tpu as pltpu

N = 10000
IN_F = 128
HID = 128
OUT_F = 64
BR = 200
NB = N // BR
NSLOT = 5


def _row_of(u):
    up = u // NB
    ui = u - up * NB
    return ui + up * (NB - 1 - 2 * ui)


def _body(x_ref, W1_ref, b1_ref, W2_ref, b2_ref, adj_ref, out_ref,
          s1_scr, s2_scr, abuf, sems):
    p = pl.program_id(0)
    i = pl.program_id(1)
    t = p * NB + i
    row = _row_of(t)
    slot = lax.rem(row, NSLOT)

    @pl.when(t == 0)
    def _prime():
        for q in range(NSLOT - 1):
            pltpu.make_async_copy(
                adj_ref.at[pl.ds(q * BR, BR), :], abuf.at[q], sems.at[q]
            ).start()

    u = t + NSLOT - 1
    urow = _row_of(u)
    uslot = lax.rem(urow, NSLOT)

    @pl.when((u < 2 * NB) & ~((u >= NB) & (urow >= NB - NSLOT)))
    def _issue():
        pltpu.make_async_copy(
            adj_ref.at[pl.ds(urow * BR, BR), :], abuf.at[uslot], sems.at[uslot]
        ).start()

    @pl.when((p == 0) | (row < NB - NSLOT))
    def _wait():
        pltpu.make_async_copy(
            adj_ref.at[pl.ds(row * BR, BR), :], abuf.at[slot], sems.at[slot]
        ).wait()

    a_ref = abuf.at[slot]

    @pl.when(p == 0)
    def _phase_a():
        @pl.when(i == 0)
        def _():
            s1_scr[...] = jnp.dot(x_ref[...], W1_ref[...],
                                  preferred_element_type=jnp.float32)

        h = jnp.dot(a_ref[...], s1_scr[...],
                    preferred_element_type=jnp.float32) + b1_ref[...]
        h = jnp.where(h >= 0, h, 0.2 * h)
        s2_scr[pl.ds(row * BR, BR), :] = jnp.dot(
            h, W2_ref[...], preferred_element_type=jnp.float32)

    @pl.when(p == 1)
    def _phase_b():
        o = jnp.dot(a_ref[...], s2_scr[...],
                    preferred_element_type=jnp.float32) + b2_ref[...]
        m = jnp.max(o, axis=1, keepdims=True)
        e = o - m
        lse = jnp.log(jnp.sum(jnp.exp(e), axis=1, keepdims=True))
        out_ref[...] = e - lse


def kernel(x, adj, W1, b1, W2, b2):
    return pl.pallas_call(
        _body,
        grid=(2, NB),
        in_specs=[
            pl.BlockSpec((N, IN_F), lambda p, i: (0, 0)),
            pl.BlockSpec((IN_F, HID), lambda p, i: (0, 0)),
            pl.BlockSpec((1, HID), lambda p, i: (0, 0)),
            pl.BlockSpec((HID, OUT_F), lambda p, i: (0, 0)),
            pl.BlockSpec((1, OUT_F), lambda p, i: (0, 0)),
            pl.BlockSpec(memory_space=pltpu.MemorySpace.HBM),
        ],
        out_specs=pl.BlockSpec((BR, OUT_F), lambda p, i: (NB - 1 - p * i, 0)),
        out_shape=jax.ShapeDtypeStruct((N, OUT_F), jnp.float32),
        scratch_shapes=[
            pltpu.VMEM((N, HID), jnp.float32),
            pltpu.VMEM((N, OUT_F), jnp.float32),
            pltpu.VMEM((NSLOT, BR, N), jnp.float32),
            pltpu.SemaphoreType.DMA((NSLOT,)),
        ],
        compiler_params=pltpu.CompilerParams(
            dimension_semantics=("arbitrary", "arbitrary"),
        ),
    )(x, W1, b1.reshape(1, HID), W2, b2.reshape(1, OUT_F), adj)


# speedup vs baseline: 1.0042x; 1.0042x over previous
import jax
import jax.numpy as jnp
from jax import lax
from jax.experimental import pallas as pl
from jax.experimental.pallas import tpu as pltpu

N = 10000
IN_F = 128
HID = 128
OUT_F = 64
BR = 200
NB = N // BR
NSLOT = 4


def _row_of(u):
    up = u // NB
    ui = u - up * NB
    return ui + up * (NB - 1 - 2 * ui)


def _body(x_ref, W1_ref, b1_ref, W2_ref, b2_ref, adj_ref, out_ref,
          s1_scr, s2_scr, abuf, sems):
    p = pl.program_id(0)
    i = pl.program_id(1)
    t = p * NB + i
    row = _row_of(t)
    slot = lax.rem(row, NSLOT)

    @pl.when(t == 0)
    def _prime():
        for q in range(NSLOT - 1):
            pltpu.make_async_copy(
                adj_ref.at[pl.ds(q * BR, BR), :], abuf.at[q], sems.at[q]
            ).start()

    u = t + NSLOT - 1
    urow = _row_of(u)
    uslot = lax.rem(urow, NSLOT)

    @pl.when((u < 2 * NB) & ~((u >= NB) & (urow >= NB - NSLOT)))
    def _issue():
        pltpu.make_async_copy(
            adj_ref.at[pl.ds(urow * BR, BR), :], abuf.at[uslot], sems.at[uslot]
        ).start()

    @pl.when((p == 0) | (row < NB - NSLOT))
    def _wait():
        pltpu.make_async_copy(
            adj_ref.at[pl.ds(row * BR, BR), :], abuf.at[slot], sems.at[slot]
        ).wait()

    a_ref = abuf.at[slot]

    @pl.when(p == 0)
    def _phase_a():
        @pl.when(i == 0)
        def _():
            s1_scr[...] = jnp.dot(x_ref[...], W1_ref[...],
                                  preferred_element_type=jnp.float32)

        h = jnp.dot(a_ref[...], s1_scr[...],
                    preferred_element_type=jnp.float32) + b1_ref[...]
        h = jnp.where(h >= 0, h, 0.2 * h)
        s2_scr[pl.ds(row * BR, BR), :] = jnp.dot(
            h, W2_ref[...], preferred_element_type=jnp.float32)

    @pl.when(p == 1)
    def _phase_b():
        o = jnp.dot(a_ref[...], s2_scr[...],
                    preferred_element_type=jnp.float32) + b2_ref[...]
        m = jnp.max(o, axis=1, keepdims=True)
        e = o - m
        lse = jnp.log(jnp.sum(jnp.exp(e), axis=1, keepdims=True))
        out_ref[...] = e - lse


def kernel(x, adj, W1, b1, W2, b2):
    return pl.pallas_call(
        _body,
        grid=(2, NB),
        in_specs=[
            pl.BlockSpec((N, IN_F), lambda p, i: (0, 0)),
            pl.BlockSpec((IN_F, HID), lambda p, i: (0, 0)),
            pl.BlockSpec((1, HID), lambda p, i: (0, 0)),
            pl.BlockSpec((HID, OUT_F), lambda p, i: (0, 0)),
            pl.BlockSpec((1, OUT_F), lambda p, i: (0, 0)),
            pl.BlockSpec(memory_space=pltpu.MemorySpace.HBM),
        ],
        out_specs=pl.BlockSpec((BR, OUT_F), lambda p, i: (NB - 1 - p * i, 0)),
        out_shape=jax.ShapeDtypeStruct((N, OUT_F), jnp.float32),
        scratch_shapes=[
            pltpu.VMEM((N, HID), jnp.float32),
            pltpu.VMEM((N, OUT_F), jnp.float32),
            pltpu.VMEM((NSLOT, BR, N), jnp.float32),
            pltpu.SemaphoreType.DMA((NSLOT,)),
        ],
        compiler_params=pltpu.CompilerParams(
            dimension_semantics=("arbitrary", "arbitrary"),
        ),
    )(x, W1, b1.reshape(1, HID), W2, b2.reshape(1, OUT_F), adj)
